# baseline (speedup 1.0000x reference)
import functools
import jax
import jax.numpy as jnp
from jax import lax
from jax.experimental import pallas as pl
from jax.experimental.pallas import tpu as pltpu
from jax.experimental.pallas import tpu_sc as plsc

BATCH = 32
P_TOTAL = 4096
D = 1100
D_SC = 1024
D_TAIL = D - D_SC
NUM_LEVELS = 256
NW = 32
P_SC = 256
P_W = P_SC // NW
NDV = D_SC // 16
P_TC = P_TOTAL - P_SC
P_BLK = 256
N_STEPS = P_TC // P_BLK


def _prep_kernel(x_ref, vw_ref, idx_ref, t_ref):
    xf = x_ref[...].astype(jnp.float32)
    idx = jnp.round(xf / 256.0 * 255.0)
    idx = jnp.clip(idx, 0, NUM_LEVELS - 1).astype(jnp.int32)
    idx_ref[...] = idx.T
    t = jnp.sum((vw_ref[...] < 0).astype(jnp.int32), axis=0, keepdims=True)
    t_ref[...] = t


def _replicate_lane(row, b):
    idxv = jnp.full((16,), b, jnp.int32)
    return row.at[idxv].get(mode="promise_in_bounds")


def _sc_body(idxt_hbm, pos_hbm, t_hbm, part_hbm, tot_hbm,
             idxt_v, pos_v, t_v, acc_v, tot_v):
    wid = lax.axis_index("s") * 2 + lax.axis_index("c")
    p0 = wid * P_W

    pltpu.sync_copy(idxt_hbm.at[pl.ds(p0 * BATCH, P_W * BATCH)], idxt_v)
    pltpu.sync_copy(t_hbm.at[0], t_v)
    pltpu.sync_copy(pos_hbm.at[pl.ds(p0, P_W), pl.ds(0, D_SC)], pos_v)

    def dv_body(dv, _):
        sl = pl.ds(dv * 16, 16)
        tv = t_v[sl]
        zero = jnp.zeros((16,), jnp.float32)
        accs = [zero for _ in range(BATCH)]
        totv = zero

        def p_body(p, carry):
            accs, totv = carry
            pv = pos_v[p, sl]
            totv = totv + pv
            base = p * BATCH
            row0 = idxt_v[pl.ds(base, 16)]
            row1 = idxt_v[pl.ds(base + 16, 16)]
            new = []
            for b in range(BATCH):
                iv = _replicate_lane(row0 if b < 16 else row1, b % 16)
                m = iv >= tv
                new.append(accs[b] + jnp.where(m, pv, zero))
            return new, totv

        accs, totv = lax.fori_loop(0, P_W, p_body, (accs, totv))
        for b in range(BATCH):
            acc_v[b, sl] = accs[b]
        tot_v[sl] = totv
        return 0

    lax.fori_loop(0, NDV, dv_body, 0)

    pltpu.sync_copy(acc_v, part_hbm.at[wid])
    pltpu.sync_copy(tot_v, tot_hbm.at[wid])


def _tc_kernel(x_ref, pos_ref, vw_ref, hv_ref, acc_ref, tot_ref, t_ref):
    i = pl.program_id(0)

    @pl.when(i == 0)
    def _init():
        acc_ref[...] = jnp.zeros_like(acc_ref)
        tot_ref[...] = jnp.zeros_like(tot_ref)
        t_ref[...] = jnp.sum((vw_ref[...] < 0).astype(jnp.int32), axis=0,
                             keepdims=True)

    pos = pos_ref[...]
    t = t_ref[0, :]

    xf = x_ref[...].astype(jnp.float32)
    idx = jnp.round(xf / 256.0 * 255.0)
    idx = jnp.clip(idx, 0, NUM_LEVELS - 1).astype(jnp.int32)

    tot_ref[...] += jnp.sum(pos, axis=0, keepdims=True)

    rows = []
    for b in range(BATCH):
        mask = idx[b, :, None] >= t[None, :]
        masked = jnp.where(mask, pos, 0.0)
        rows.append(jnp.sum(masked, axis=0))
    acc_ref[...] += jnp.stack(rows, axis=0)

    @pl.when(i == N_STEPS - 1)
    def _fin():
        hv_ref[...] = 2.0 * acc_ref[...] - tot_ref[...]


def _combine_kernel(part_ref, tot_ref, hv_ref, x_ref, post_ref, vwt_ref,
                    out_ref):
    hv_tc = hv_ref[...]
    s = jnp.sum(part_ref[...], axis=0)
    tot = jnp.sum(tot_ref[...], axis=0)
    hv_main = 2.0 * s - tot + hv_tc[:, :D_SC]
    out_ref[:, :D_SC] = jnp.where(hv_main > 0, 1.0, -1.0)

    xf = x_ref[...].astype(jnp.float32)
    idx = jnp.round(xf / 256.0 * 255.0)
    idx = jnp.clip(idx, 0, NUM_LEVELS - 1).astype(jnp.int32)
    t_tail = jnp.sum((vwt_ref[...] < 0).astype(jnp.int32), axis=0)
    post = post_ref[...]
    rows = []
    for b in range(BATCH):
        mask = idx[b, :, None] >= t_tail[None, :]
        masked = jnp.where(mask, post, 0.0)
        rows.append(jnp.sum(masked, axis=0))
    s_tail = jnp.stack(rows, axis=0)
    tot_tail = jnp.sum(post, axis=0)
    hv_tail = 2.0 * s_tail - tot_tail + hv_tc[:, D_SC:]
    out_ref[:, D_SC:] = jnp.where(hv_tail > 0, 1.0, -1.0)


def kernel(x, position_weight, value_weight):
    B = x.shape[0]
    x_flat = x.reshape(B, -1)
    x_sc = x_flat[:, :P_SC]
    pos_tail = lax.slice(position_weight, (0, D_SC), (P_SC, D))
    vw_tail = lax.slice(value_weight, (0, D_SC), (NUM_LEVELS, D))

    idxt, t_sc = pl.pallas_call(
        _prep_kernel,
        out_shape=(
            jax.ShapeDtypeStruct((P_SC, B), jnp.int32),
            jax.ShapeDtypeStruct((1, D_SC), jnp.int32),
        ),
    )(x_sc, value_weight[:, :D_SC])
    idxt_flat = idxt.reshape(-1)

    mesh = plsc.VectorSubcoreMesh(core_axis_name="c", subcore_axis_name="s")
    sc = functools.partial(
        pl.kernel,
        mesh=mesh,
        out_type=(
            jax.ShapeDtypeStruct((NW, BATCH, D_SC), jnp.float32),
            jax.ShapeDtypeStruct((NW, D_SC), jnp.float32),
        ),
        scratch_types=[
            pltpu.VMEM((P_W * BATCH,), jnp.int32),
            pltpu.VMEM((P_W, D_SC), jnp.float32),
            pltpu.VMEM((D_SC,), jnp.int32),
            pltpu.VMEM((BATCH, D_SC), jnp.float32),
            pltpu.VMEM((D_SC,), jnp.float32),
        ],
    )(_sc_body)
    part, tot = sc(idxt_flat, position_weight, t_sc)

    hv_tc = pl.pallas_call(
        _tc_kernel,
        grid=(N_STEPS,),
        in_specs=[
            pl.BlockSpec((BATCH, P_BLK), lambda i: (0, i + P_SC // P_BLK)),
            pl.BlockSpec((P_BLK, D), lambda i: (i + P_SC // P_BLK, 0)),
            pl.BlockSpec((NUM_LEVELS, D), lambda i: (0, 0)),
        ],
        out_specs=pl.BlockSpec((BATCH, D), lambda i: (0, 0)),
        out_shape=jax.ShapeDtypeStruct((BATCH, D), jnp.float32),
        scratch_shapes=[
            pltpu.VMEM((BATCH, D), jnp.float32),
            pltpu.VMEM((1, D), jnp.float32),
            pltpu.VMEM((1, D), jnp.int32),
        ],
    )(x_flat, position_weight, value_weight)

    out = pl.pallas_call(
        _combine_kernel,
        out_shape=jax.ShapeDtypeStruct((BATCH, D), jnp.float32),
    )(part, tot, hv_tc, x_sc, pos_tail, vw_tail)
    return out


# speedup vs baseline: 1.0499x; 1.0499x over previous
import functools
import jax
import jax.numpy as jnp
from jax import lax
from jax.experimental import pallas as pl
from jax.experimental.pallas import tpu as pltpu
from jax.experimental.pallas import tpu_sc as plsc

BATCH = 32
P_TOTAL = 4096
D = 1100
D_SC = 1024
D_TAIL = D - D_SC
NUM_LEVELS = 256
NW = 32
P_SC = 768
P_W = P_SC // NW
NDV = D_SC // 16
P_TC = P_TOTAL - P_SC
P_BLK = 256
N_STEPS = P_TC // P_BLK


def _prep_kernel(x_ref, vw_ref, idx_ref, t_ref):
    xf = x_ref[...].astype(jnp.float32)
    idx = jnp.round(xf / 256.0 * 255.0)
    idx = jnp.clip(idx, 0, NUM_LEVELS - 1).astype(jnp.int32)
    idx_ref[...] = idx.T
    t = jnp.sum((vw_ref[...] < 0).astype(jnp.int32), axis=0, keepdims=True)
    t_ref[...] = t


def _replicate_lane(row, b):
    idxv = jnp.full((16,), b, jnp.int32)
    return row.at[idxv].get(mode="promise_in_bounds")


def _sc_body(idxt_hbm, pos_hbm, t_hbm, part_hbm, tot_hbm,
             idxt_v, pos_v, t_v, acc_v, tot_v):
    wid = lax.axis_index("s") * 2 + lax.axis_index("c")
    p0 = wid * P_W

    pltpu.sync_copy(idxt_hbm.at[pl.ds(p0 * BATCH, P_W * BATCH)], idxt_v)
    pltpu.sync_copy(t_hbm.at[0], t_v)
    pltpu.sync_copy(pos_hbm.at[pl.ds(p0, P_W), pl.ds(0, D_SC)], pos_v)

    def dv_body(dv, _):
        sl = pl.ds(dv * 16, 16)
        tv = t_v[sl]
        zero = jnp.zeros((16,), jnp.float32)
        accs = [zero for _ in range(BATCH)]
        totv = zero

        def p_body(p, carry):
            accs, totv = carry
            pv = pos_v[p, sl]
            totv = totv + pv
            base = p * BATCH
            row0 = idxt_v[pl.ds(base, 16)]
            row1 = idxt_v[pl.ds(base + 16, 16)]
            new = []
            for b in range(BATCH):
                iv = _replicate_lane(row0 if b < 16 else row1, b % 16)
                m = iv >= tv
                new.append(accs[b] + jnp.where(m, pv, zero))
            return new, totv

        accs, totv = lax.fori_loop(0, P_W, p_body, (accs, totv))
        for b in range(BATCH):
            acc_v[b, sl] = accs[b]
        tot_v[sl] = totv
        return 0

    lax.fori_loop(0, NDV, dv_body, 0)

    pltpu.sync_copy(acc_v, part_hbm.at[wid])
    pltpu.sync_copy(tot_v, tot_hbm.at[wid])


def _tc_kernel(x_ref, pos_ref, vw_ref, hv_ref, acc_ref, tot_ref, t_ref):
    i = pl.program_id(0)

    @pl.when(i == 0)
    def _init():
        acc_ref[...] = jnp.zeros_like(acc_ref)
        tot_ref[...] = jnp.zeros_like(tot_ref)
        t_ref[...] = jnp.sum((vw_ref[...] < 0).astype(jnp.int32), axis=0,
                             keepdims=True)

    pos = pos_ref[...]
    t = t_ref[0, :]

    xf = x_ref[...].astype(jnp.float32)
    idx = jnp.round(xf / 256.0 * 255.0)
    idx = jnp.clip(idx, 0, NUM_LEVELS - 1).astype(jnp.int32)

    tot_ref[...] += jnp.sum(pos, axis=0, keepdims=True)

    rows = []
    for b in range(BATCH):
        mask = idx[b, :, None] >= t[None, :]
        masked = jnp.where(mask, pos, 0.0)
        rows.append(jnp.sum(masked, axis=0))
    acc_ref[...] += jnp.stack(rows, axis=0)

    @pl.when(i == N_STEPS - 1)
    def _fin():
        hv_ref[...] = 2.0 * acc_ref[...] - tot_ref[...]


def _combine_kernel(part_ref, tot_ref, hv_ref, x_ref, post_ref, vwt_ref,
                    out_ref):
    hv_tc = hv_ref[...]
    s = jnp.sum(part_ref[...], axis=0)
    tot = jnp.sum(tot_ref[...], axis=0)
    hv_main = 2.0 * s - tot + hv_tc[:, :D_SC]
    out_ref[:, :D_SC] = jnp.where(hv_main > 0, 1.0, -1.0)

    xf = x_ref[...].astype(jnp.float32)
    idx = jnp.round(xf / 256.0 * 255.0)
    idx = jnp.clip(idx, 0, NUM_LEVELS - 1).astype(jnp.int32)
    t_tail = jnp.sum((vwt_ref[...] < 0).astype(jnp.int32), axis=0)
    post = post_ref[...]
    rows = []
    for b in range(BATCH):
        mask = idx[b, :, None] >= t_tail[None, :]
        masked = jnp.where(mask, post, 0.0)
        rows.append(jnp.sum(masked, axis=0))
    s_tail = jnp.stack(rows, axis=0)
    tot_tail = jnp.sum(post, axis=0)
    hv_tail = 2.0 * s_tail - tot_tail + hv_tc[:, D_SC:]
    out_ref[:, D_SC:] = jnp.where(hv_tail > 0, 1.0, -1.0)


def kernel(x, position_weight, value_weight):
    B = x.shape[0]
    x_flat = x.reshape(B, -1)
    x_sc = x_flat[:, :P_SC]
    pos_tail = lax.slice(position_weight, (0, D_SC), (P_SC, D))
    vw_tail = lax.slice(value_weight, (0, D_SC), (NUM_LEVELS, D))

    idxt, t_sc = pl.pallas_call(
        _prep_kernel,
        out_shape=(
            jax.ShapeDtypeStruct((P_SC, B), jnp.int32),
            jax.ShapeDtypeStruct((1, D_SC), jnp.int32),
        ),
    )(x_sc, value_weight[:, :D_SC])
    idxt_flat = idxt.reshape(-1)

    mesh = plsc.VectorSubcoreMesh(core_axis_name="c", subcore_axis_name="s")
    sc = functools.partial(
        pl.kernel,
        mesh=mesh,
        out_type=(
            jax.ShapeDtypeStruct((NW, BATCH, D_SC), jnp.float32),
            jax.ShapeDtypeStruct((NW, D_SC), jnp.float32),
        ),
        scratch_types=[
            pltpu.VMEM((P_W * BATCH,), jnp.int32),
            pltpu.VMEM((P_W, D_SC), jnp.float32),
            pltpu.VMEM((D_SC,), jnp.int32),
            pltpu.VMEM((BATCH, D_SC), jnp.float32),
            pltpu.VMEM((D_SC,), jnp.float32),
        ],
    )(_sc_body)
    part, tot = sc(idxt_flat, position_weight, t_sc)

    hv_tc = pl.pallas_call(
        _tc_kernel,
        grid=(N_STEPS,),
        in_specs=[
            pl.BlockSpec((BATCH, P_BLK), lambda i: (0, i + P_SC // P_BLK)),
            pl.BlockSpec((P_BLK, D), lambda i: (i + P_SC // P_BLK, 0)),
            pl.BlockSpec((NUM_LEVELS, D), lambda i: (0, 0)),
        ],
        out_specs=pl.BlockSpec((BATCH, D), lambda i: (0, 0)),
        out_shape=jax.ShapeDtypeStruct((BATCH, D), jnp.float32),
        scratch_shapes=[
            pltpu.VMEM((BATCH, D), jnp.float32),
            pltpu.VMEM((1, D), jnp.float32),
            pltpu.VMEM((1, D), jnp.int32),
        ],
    )(x_flat, position_weight, value_weight)

    out = pl.pallas_call(
        _combine_kernel,
        out_shape=jax.ShapeDtypeStruct((BATCH, D), jnp.float32),
    )(part, tot, hv_tc, x_sc, pos_tail, vw_tail)
    return out
